# baseline (speedup 1.0000x reference)
import jax
import jax.numpy as jnp
from jax import lax
from jax.experimental import pallas as pl
from jax.experimental.pallas import tpu as pltpu
from jax.experimental.pallas import tpu_sc as plsc

BATCH = 4096
SEQ_LEN = 200
HIDDEN = 64
LANES = 16

NUM_CORES = 2
NUM_SUBCORES = 16
NUM_WORKERS = NUM_CORES * NUM_SUBCORES

LT = SEQ_LEN // 8

_TAKE_DNUMS = lax.GatherDimensionNumbers(
    offset_dims=(), collapsed_slice_dims=(0,), start_index_map=(0,)
)


def _take16(vec, idx):
    return lax.gather(
        vec,
        idx[:, None],
        _TAKE_DNUMS,
        slice_sizes=(1,),
        mode=lax.GatherScatterMode.PROMISE_IN_BOUNDS,
    )


def _body(xt_hbm, wt2_hbm, pt_hbm, out_hbm, idx_v, idx2_v, pos_v,
          rows0, rows1, st0, st1, semg0, semg1, semo0, semo1):
    wid = lax.axis_index("s") * NUM_CORES + lax.axis_index("c")
    b0 = wid * 128

    rows = (rows0, rows1)
    stage = (st0, st1)
    semg = (semg0, semg1)
    semo = (semo0, semo1)

    pltpu.sync_copy(pt_hbm.at[:, pl.ds(0, 256)], pos_v)

    iota = lax.iota(jnp.int32, LANES)
    rowvec = [iota + (16 * k) for k in range(8)]

    def issue_gather(j, b):
        pltpu.async_copy(wt2_hbm.at[idx2_v.at[j]], rows[b], semg[b])

    def wait_gather(b):
        pltpu.make_async_copy(wt2_hbm.at[pl.ds(0, 128)], rows[b], semg[b]).wait()

    def wait_out(b):
        pltpu.make_async_copy(stage[b], out_hbm.at[0, :, pl.ds(b0, 128)], semo[b]).wait()

    def lt_body(lt, carry):
        l0 = lt * 8
        lbase = 16 * (lt // 2)
        lane_off = 8 * (lt % 2)
        pltpu.sync_copy(xt_hbm.at[pl.ds(l0, 8), pl.ds(b0, 128)], idx_v)
        for j in range(8):
            for k in range(8):
                v16 = idx_v[j, pl.ds(16 * k, 16)]
                idx2_v[j, pl.ds(16 * k, 16)] = lax.shift_right_logical(v16, 1)
        issue_gather(0, 0)
        for j in range(8):
            if j < 7:
                issue_gather(j + 1, (j + 1) % 2)
            b = j % 2
            wait_gather(b)
            if j >= 2:
                wait_out(b)
            else:
                @pl.when(lt > 0)
                def _():
                    wait_out(b)
            pv = [
                lax.shift_left(idx_v[j, pl.ds(16 * k, 16)] & 1, 6)
                for k in range(8)
            ]

            lane_sel = jnp.full((LANES,), lane_off + j, dtype=jnp.int32)

            def h_body(h, c):
                prow = pos_v[h, pl.ds(lbase, 16)]
                ps = _take16(prow, lane_sel)
                for k in range(8):
                    col = pv[k] + h
                    vec = plsc.load_gather(rows[b], [rowvec[k], col])
                    stage[b][h, pl.ds(16 * k, 16)] = vec + ps
                return c

            lax.fori_loop(0, HIDDEN, h_body, 0)
            pltpu.async_copy(stage[b], out_hbm.at[l0 + j, :, pl.ds(b0, 128)], semo[b])
        return carry

    lax.fori_loop(0, LT, lt_body, 0)
    wait_out(0)
    wait_out(1)


@jax.jit
def _run(xt, wt2, pt):
    mesh = plsc.VectorSubcoreMesh(core_axis_name="c", subcore_axis_name="s")
    return pl.kernel(
        _body,
        out_type=jax.ShapeDtypeStruct((SEQ_LEN, HIDDEN, BATCH), jnp.float32),
        mesh=mesh,
        compiler_params=pltpu.CompilerParams(
            use_tc_tiling_on_sc=True, needs_layout_passes=False
        ),
        scratch_types=[
            pltpu.VMEM((8, 128), jnp.int32),
            pltpu.VMEM((8, 128), jnp.int32),
            pltpu.VMEM((HIDDEN, 256), jnp.float32),
            pltpu.VMEM((128, 128), jnp.float32),
            pltpu.VMEM((128, 128), jnp.float32),
            pltpu.VMEM((HIDDEN, 128), jnp.float32),
            pltpu.VMEM((HIDDEN, 128), jnp.float32),
            pltpu.SemaphoreType.DMA,
            pltpu.SemaphoreType.DMA,
            pltpu.SemaphoreType.DMA,
            pltpu.SemaphoreType.DMA,
        ],
    )(xt, wt2, pt)


def kernel(x, word_table, pos_table):
    xt = x.astype(jnp.int32).T
    wt2 = word_table.reshape(500000, 128)
    pt = pos_table.T
    out_t = _run(xt, wt2, pt)
    return jnp.transpose(out_t, (2, 0, 1))


# speedup vs baseline: 1.5236x; 1.5236x over previous
import jax
import jax.numpy as jnp
from jax import lax
from jax.experimental import pallas as pl
from jax.experimental.pallas import tpu as pltpu
from jax.experimental.pallas import tpu_sc as plsc

BATCH = 4096
SEQ_LEN = 200
HIDDEN = 64
LANES = 16

NUM_CORES = 2
NUM_SUBCORES = 16
NUM_WORKERS = NUM_CORES * NUM_SUBCORES

LT = SEQ_LEN // 8

_TAKE_DNUMS = lax.GatherDimensionNumbers(
    offset_dims=(), collapsed_slice_dims=(0,), start_index_map=(0,)
)


def _take16(vec, idx):
    return lax.gather(
        vec,
        idx[:, None],
        _TAKE_DNUMS,
        slice_sizes=(1,),
        mode=lax.GatherScatterMode.PROMISE_IN_BOUNDS,
    )


def _body(xt_hbm, wt2_hbm, pt_hbm, out_hbm, idx_v, idx2_v, pos_v,
          rows0, rows1, st0, st1, semg0, semg1, semo0, semo1):
    wid = lax.axis_index("s") * NUM_CORES + lax.axis_index("c")
    b0 = wid * 128

    rows = (rows0, rows1)
    stage = (st0, st1)
    semg = (semg0, semg1)
    semo = (semo0, semo1)

    pltpu.sync_copy(pt_hbm.at[:, pl.ds(0, 256)], pos_v)

    iota = lax.iota(jnp.int32, LANES)
    rowvec = [iota + (16 * k) for k in range(8)]

    def issue_gather(j, b):
        pltpu.async_copy(wt2_hbm.at[idx2_v.at[j]], rows[b], semg[b])

    def wait_gather(b):
        pltpu.make_async_copy(wt2_hbm.at[pl.ds(0, 128)], rows[b], semg[b]).wait()

    def wait_out(b):
        pltpu.make_async_copy(stage[b], out_hbm.at[0, :, pl.ds(b0, 128)], semo[b]).wait()

    def lt_body(lt, carry):
        l0 = lt * 8
        lbase = 16 * (lt // 2)
        lane_off = 8 * (lt % 2)
        pltpu.sync_copy(xt_hbm.at[pl.ds(l0, 8), pl.ds(b0, 128)], idx_v)
        for j in range(8):
            for k in range(8):
                v16 = idx_v[j, pl.ds(16 * k, 16)]
                idx2_v[j, pl.ds(16 * k, 16)] = lax.shift_right_logical(v16, 1)
        issue_gather(0, 0)
        for j in range(8):
            if j < 7:
                issue_gather(j + 1, (j + 1) % 2)
            b = j % 2
            wait_gather(b)
            if j >= 2:
                wait_out(b)
            else:
                @pl.when(lt > 0)
                def _():
                    wait_out(b)
            pv = [
                lax.shift_left(idx_v[j, pl.ds(16 * k, 16)] & 1, 6)
                for k in range(8)
            ]

            lane_sel = jnp.full((LANES,), lane_off + j, dtype=jnp.int32)

            @plsc.parallel_loop(0, HIDDEN, unroll=4)
            def _h_loop(h):
                prow = pos_v[h, pl.ds(lbase, 16)]
                ps = _take16(prow, lane_sel)
                for k in range(8):
                    col = pv[k] + h
                    vec = plsc.load_gather(rows[b], [rowvec[k], col])
                    stage[b][h, pl.ds(16 * k, 16)] = vec + ps
            pltpu.async_copy(stage[b], out_hbm.at[l0 + j, :, pl.ds(b0, 128)], semo[b])
        return carry

    lax.fori_loop(0, LT, lt_body, 0)
    wait_out(0)
    wait_out(1)


@jax.jit
def _run(xt, wt2, pt):
    mesh = plsc.VectorSubcoreMesh(core_axis_name="c", subcore_axis_name="s")
    return pl.kernel(
        _body,
        out_type=jax.ShapeDtypeStruct((SEQ_LEN, HIDDEN, BATCH), jnp.float32),
        mesh=mesh,
        compiler_params=pltpu.CompilerParams(
            use_tc_tiling_on_sc=True, needs_layout_passes=False
        ),
        scratch_types=[
            pltpu.VMEM((8, 128), jnp.int32),
            pltpu.VMEM((8, 128), jnp.int32),
            pltpu.VMEM((HIDDEN, 256), jnp.float32),
            pltpu.VMEM((128, 128), jnp.float32),
            pltpu.VMEM((128, 128), jnp.float32),
            pltpu.VMEM((HIDDEN, 128), jnp.float32),
            pltpu.VMEM((HIDDEN, 128), jnp.float32),
            pltpu.SemaphoreType.DMA,
            pltpu.SemaphoreType.DMA,
            pltpu.SemaphoreType.DMA,
            pltpu.SemaphoreType.DMA,
        ],
    )(xt, wt2, pt)


def kernel(x, word_table, pos_table):
    xt = x.astype(jnp.int32).T
    wt2 = word_table.reshape(500000, 128)
    pt = pos_table.T
    out_t = _run(xt, wt2, pt)
    return jnp.transpose(out_t, (2, 0, 1))


# speedup vs baseline: 2.2455x; 1.4738x over previous
import jax
import jax.numpy as jnp
from jax import lax
from jax.experimental import pallas as pl
from jax.experimental.pallas import tpu as pltpu
from jax.experimental.pallas import tpu_sc as plsc

BATCH = 4096
SEQ_LEN = 200
HIDDEN = 64
LANES = 16

NUM_CORES = 2
NUM_SUBCORES = 16
NUM_WORKERS = NUM_CORES * NUM_SUBCORES

LT = SEQ_LEN // 8

_TAKE_DNUMS = lax.GatherDimensionNumbers(
    offset_dims=(), collapsed_slice_dims=(0,), start_index_map=(0,)
)


def _take16(vec, idx):
    return lax.gather(
        vec,
        idx[:, None],
        _TAKE_DNUMS,
        slice_sizes=(1,),
        mode=lax.GatherScatterMode.PROMISE_IN_BOUNDS,
    )


def _body(xt_hbm, wt2_hbm, px_hbm, out_hbm, idx_v, idx2_v, poslt_v,
          rows0, rows1, st0, st1, semg0, semg1, semo0, semo1):
    wid = lax.axis_index("s") * NUM_CORES + lax.axis_index("c")
    b0 = wid * 128

    rows = (rows0, rows1)
    stage = (st0, st1)
    semg = (semg0, semg1)
    semo = (semo0, semo1)

    iota = lax.iota(jnp.int32, LANES)
    rowvec = [iota + (16 * k) for k in range(8)]

    def issue_gather(j, b):
        pltpu.async_copy(wt2_hbm.at[idx2_v.at[j]], rows[b], semg[b])

    def wait_gather(b):
        pltpu.make_async_copy(wt2_hbm.at[pl.ds(0, 128)], rows[b], semg[b]).wait()

    def wait_out(b):
        pltpu.make_async_copy(stage[b], out_hbm.at[0, :, pl.ds(b0, 128)], semo[b]).wait()

    def lt_body(lt, carry):
        l0 = lt * 8
        pltpu.sync_copy(xt_hbm.at[pl.ds(l0, 8), pl.ds(b0, 128)], idx_v)
        pltpu.sync_copy(px_hbm.at[pl.ds(l0, 8)], poslt_v)
        for j in range(8):
            for k in range(8):
                v16 = idx_v[j, pl.ds(16 * k, 16)]
                idx2_v[j, pl.ds(16 * k, 16)] = lax.shift_right_logical(v16, 1)
        issue_gather(0, 0)
        for j in range(8):
            if j < 7:
                issue_gather(j + 1, (j + 1) % 2)
            b = j % 2
            wait_gather(b)
            if j >= 2:
                wait_out(b)
            else:
                @pl.when(lt > 0)
                def _():
                    wait_out(b)
            pv = [
                lax.shift_left(idx_v[j, pl.ds(16 * k, 16)] & 1, 6)
                for k in range(8)
            ]

            @plsc.parallel_loop(0, HIDDEN, unroll=2)
            def _diag_loop(i):
                d = i & 15
                h0 = lax.shift_right_logical(i, 4) * 16
                rot = (iota + d) & 15
                hvec = rot + h0
                povec = poslt_v[j, pl.ds(h0, 16)]
                ps = _take16(povec, rot)
                for k in range(8):
                    col = pv[k] + hvec
                    vec = plsc.load_gather(rows[b], [rowvec[k], col])
                    plsc.store_scatter(stage[b], [hvec, rowvec[k]], vec + ps)
            pltpu.async_copy(stage[b], out_hbm.at[l0 + j, :, pl.ds(b0, 128)], semo[b])
        return carry

    lax.fori_loop(0, LT, lt_body, 0)
    wait_out(0)
    wait_out(1)


@jax.jit
def _run(xt, wt2, pt):
    mesh = plsc.VectorSubcoreMesh(core_axis_name="c", subcore_axis_name="s")
    return pl.kernel(
        _body,
        out_type=jax.ShapeDtypeStruct((SEQ_LEN, HIDDEN, BATCH), jnp.float32),
        mesh=mesh,
        compiler_params=pltpu.CompilerParams(
            use_tc_tiling_on_sc=True, needs_layout_passes=False
        ),
        scratch_types=[
            pltpu.VMEM((8, 128), jnp.int32),
            pltpu.VMEM((8, 128), jnp.int32),
            pltpu.VMEM((8, 128), jnp.float32),
            pltpu.VMEM((128, 128), jnp.float32),
            pltpu.VMEM((128, 128), jnp.float32),
            pltpu.VMEM((HIDDEN, 128), jnp.float32),
            pltpu.VMEM((HIDDEN, 128), jnp.float32),
            pltpu.SemaphoreType.DMA,
            pltpu.SemaphoreType.DMA,
            pltpu.SemaphoreType.DMA,
            pltpu.SemaphoreType.DMA,
        ],
    )(xt, wt2, pt)


def kernel(x, word_table, pos_table):
    xt = x.astype(jnp.int32).T
    wt2 = word_table.reshape(500000, 128)
    pos200 = pos_table[:SEQ_LEN]
    posx = jnp.concatenate([pos200, pos200], axis=1)
    out_t = _run(xt, wt2, posx)
    return jnp.transpose(out_t, (2, 0, 1))


# speedup vs baseline: 2.3350x; 1.0399x over previous
import jax
import jax.numpy as jnp
from jax import lax
from jax.experimental import pallas as pl
from jax.experimental.pallas import tpu as pltpu
from jax.experimental.pallas import tpu_sc as plsc

BATCH = 4096
SEQ_LEN = 200
HIDDEN = 64
LANES = 16

NUM_CORES = 2
NUM_SUBCORES = 16
NUM_WORKERS = NUM_CORES * NUM_SUBCORES

LT = SEQ_LEN // 8

_TAKE_DNUMS = lax.GatherDimensionNumbers(
    offset_dims=(), collapsed_slice_dims=(0,), start_index_map=(0,)
)


def _take16(vec, idx):
    return lax.gather(
        vec,
        idx[:, None],
        _TAKE_DNUMS,
        slice_sizes=(1,),
        mode=lax.GatherScatterMode.PROMISE_IN_BOUNDS,
    )


def _body(xt_hbm, wt2_hbm, px_hbm, out_hbm, idx_v, idx2_v, posall_v,
          rows0, rows1, rows2, rows3, st0, st1,
          semg0, semg1, semg2, semg3, semo0, semo1):
    wid = lax.axis_index("s") * NUM_CORES + lax.axis_index("c")
    b0 = wid * 128

    rows = (rows0, rows1, rows2, rows3)
    stage = (st0, st1)
    semg = (semg0, semg1, semg2, semg3)
    semo = (semo0, semo1)

    iota = lax.iota(jnp.int32, LANES)
    rowvec = [iota + (16 * k) for k in range(8)]

    def issue_gather(j, b):
        pltpu.async_copy(wt2_hbm.at[idx2_v.at[j]], rows[b], semg[b])

    def wait_gather(b):
        pltpu.make_async_copy(wt2_hbm.at[pl.ds(0, 128)], rows[b], semg[b]).wait()

    def wait_out(b):
        pltpu.make_async_copy(stage[b], out_hbm.at[0, :, pl.ds(b0, 128)], semo[b]).wait()

    pltpu.sync_copy(px_hbm, posall_v)

    def lt_body(lt, carry):
        l0 = lt * 8
        pltpu.sync_copy(xt_hbm.at[pl.ds(l0, 8), pl.ds(b0, 128)], idx_v)
        for j in range(8):
            for k in range(8):
                v16 = idx_v[j, pl.ds(16 * k, 16)]
                idx2_v[j, pl.ds(16 * k, 16)] = lax.shift_right_logical(v16, 1)
        for j3 in range(3):
            issue_gather(j3, j3)
        for j in range(8):
            if j < 5:
                issue_gather(j + 3, (j + 3) % 4)
            b = j % 4
            s = j % 2
            wait_gather(b)
            if j >= 2:
                wait_out(s)
            else:
                @pl.when(lt > 0)
                def _():
                    wait_out(s)
            pv = [
                lax.shift_left(idx_v[j, pl.ds(16 * k, 16)] & 1, 6)
                for k in range(8)
            ]

            @plsc.parallel_loop(0, HIDDEN, unroll=2)
            def _diag_loop(i):
                d = i & 15
                h0 = lax.shift_right_logical(i, 4) * 16
                rot = (iota + d) & 15
                hvec = rot + h0
                povec = posall_v[l0 + j, pl.ds(h0, 16)]
                ps = _take16(povec, rot)
                for k in range(8):
                    col = pv[k] + hvec
                    vec = plsc.load_gather(rows[b], [rowvec[k], col])
                    plsc.store_scatter(stage[s], [hvec, rowvec[k]], vec + ps)
            pltpu.async_copy(stage[s], out_hbm.at[l0 + j, :, pl.ds(b0, 128)], semo[s])
        return carry

    lax.fori_loop(0, LT, lt_body, 0)
    wait_out(0)
    wait_out(1)


@jax.jit
def _run(xt, wt2, pt):
    mesh = plsc.VectorSubcoreMesh(core_axis_name="c", subcore_axis_name="s")
    return pl.kernel(
        _body,
        out_type=jax.ShapeDtypeStruct((SEQ_LEN, HIDDEN, BATCH), jnp.float32),
        mesh=mesh,
        compiler_params=pltpu.CompilerParams(
            use_tc_tiling_on_sc=True, needs_layout_passes=False
        ),
        scratch_types=[
            pltpu.VMEM((8, 128), jnp.int32),
            pltpu.VMEM((8, 128), jnp.int32),
            pltpu.VMEM((SEQ_LEN, 128), jnp.float32),
            pltpu.VMEM((128, 128), jnp.float32),
            pltpu.VMEM((128, 128), jnp.float32),
            pltpu.VMEM((128, 128), jnp.float32),
            pltpu.VMEM((128, 128), jnp.float32),
            pltpu.VMEM((HIDDEN, 128), jnp.float32),
            pltpu.VMEM((HIDDEN, 128), jnp.float32),
            pltpu.SemaphoreType.DMA,
            pltpu.SemaphoreType.DMA,
            pltpu.SemaphoreType.DMA,
            pltpu.SemaphoreType.DMA,
            pltpu.SemaphoreType.DMA,
            pltpu.SemaphoreType.DMA,
        ],
    )(xt, wt2, pt)


def kernel(x, word_table, pos_table):
    xt = x.astype(jnp.int32).T
    wt2 = word_table.reshape(500000, 128)
    pos200 = pos_table[:SEQ_LEN]
    posx = jnp.concatenate([pos200, pos200], axis=1)
    out_t = _run(xt, wt2, posx)
    return jnp.transpose(out_t, (2, 0, 1))


# speedup vs baseline: 2.3396x; 1.0020x over previous
import jax
import jax.numpy as jnp
from jax import lax
from jax.experimental import pallas as pl
from jax.experimental.pallas import tpu as pltpu
from jax.experimental.pallas import tpu_sc as plsc

BATCH = 4096
SEQ_LEN = 200
HIDDEN = 64
LANES = 16

NUM_CORES = 2
NUM_SUBCORES = 16
NUM_WORKERS = NUM_CORES * NUM_SUBCORES

LT = SEQ_LEN // 8

_TAKE_DNUMS = lax.GatherDimensionNumbers(
    offset_dims=(), collapsed_slice_dims=(0,), start_index_map=(0,)
)


def _take16(vec, idx):
    return lax.gather(
        vec,
        idx[:, None],
        _TAKE_DNUMS,
        slice_sizes=(1,),
        mode=lax.GatherScatterMode.PROMISE_IN_BOUNDS,
    )


def _body(xt_hbm, wt2_hbm, px_hbm, out_hbm, idx_v, idx2_v, posall_v,
          rows0, rows1, rows2, rows3, st0, st1,
          semg0, semg1, semg2, semg3, semo0, semo1):
    wid = lax.axis_index("s") * NUM_CORES + lax.axis_index("c")
    b0 = wid * 128

    rows = (rows0, rows1, rows2, rows3)
    stage = (st0, st1)
    semg = (semg0, semg1, semg2, semg3)
    semo = (semo0, semo1)

    iota = lax.iota(jnp.int32, LANES)
    rowvec = [iota + (16 * k) for k in range(8)]

    def issue_gather(j, b):
        pltpu.async_copy(wt2_hbm.at[idx2_v.at[j]], rows[b], semg[b])

    def wait_gather(b):
        pltpu.make_async_copy(wt2_hbm.at[pl.ds(0, 128)], rows[b], semg[b]).wait()

    def wait_out(b):
        pltpu.make_async_copy(stage[b], out_hbm.at[0, :, pl.ds(b0, 128)], semo[b]).wait()

    pltpu.sync_copy(px_hbm, posall_v)

    def lt_body(lt, carry):
        l0 = lt * 8
        pltpu.sync_copy(xt_hbm.at[pl.ds(l0, 8), pl.ds(b0, 128)], idx_v)
        for j in range(8):
            for k in range(8):
                v16 = idx_v[j, pl.ds(16 * k, 16)]
                idx2_v[j, pl.ds(16 * k, 16)] = lax.shift_right_logical(v16, 1)
        for j3 in range(3):
            issue_gather(j3, j3)
        for j in range(8):
            if j < 5:
                issue_gather(j + 3, (j + 3) % 4)
            b = j % 4
            s = j % 2
            wait_gather(b)
            if j >= 2:
                wait_out(s)
            else:
                @pl.when(lt > 0)
                def _():
                    wait_out(s)
            pv = [
                lax.shift_left(idx_v[j, pl.ds(16 * k, 16)] & 1, 6)
                for k in range(8)
            ]

            @plsc.parallel_loop(0, HIDDEN, unroll=4)
            def _diag_loop(i):
                d = i & 15
                h0 = lax.shift_right_logical(i, 4) * 16
                rot = (iota + d) & 15
                hvec = rot + h0
                povec = posall_v[l0 + j, pl.ds(h0, 16)]
                ps = _take16(povec, rot)
                for k in range(8):
                    col = pv[k] + hvec
                    vec = plsc.load_gather(rows[b], [rowvec[k], col])
                    plsc.store_scatter(stage[s], [hvec, rowvec[k]], vec + ps)
            pltpu.async_copy(stage[s], out_hbm.at[l0 + j, :, pl.ds(b0, 128)], semo[s])
        return carry

    lax.fori_loop(0, LT, lt_body, 0)
    wait_out(0)
    wait_out(1)


@jax.jit
def _run(xt, wt2, pt):
    mesh = plsc.VectorSubcoreMesh(core_axis_name="c", subcore_axis_name="s")
    return pl.kernel(
        _body,
        out_type=jax.ShapeDtypeStruct((SEQ_LEN, HIDDEN, BATCH), jnp.float32),
        mesh=mesh,
        compiler_params=pltpu.CompilerParams(
            use_tc_tiling_on_sc=True, needs_layout_passes=False
        ),
        scratch_types=[
            pltpu.VMEM((8, 128), jnp.int32),
            pltpu.VMEM((8, 128), jnp.int32),
            pltpu.VMEM((SEQ_LEN, 128), jnp.float32),
            pltpu.VMEM((128, 128), jnp.float32),
            pltpu.VMEM((128, 128), jnp.float32),
            pltpu.VMEM((128, 128), jnp.float32),
            pltpu.VMEM((128, 128), jnp.float32),
            pltpu.VMEM((HIDDEN, 128), jnp.float32),
            pltpu.VMEM((HIDDEN, 128), jnp.float32),
            pltpu.SemaphoreType.DMA,
            pltpu.SemaphoreType.DMA,
            pltpu.SemaphoreType.DMA,
            pltpu.SemaphoreType.DMA,
            pltpu.SemaphoreType.DMA,
            pltpu.SemaphoreType.DMA,
        ],
    )(xt, wt2, pt)


def kernel(x, word_table, pos_table):
    xt = x.astype(jnp.int32).T
    wt2 = word_table.reshape(500000, 128)
    pos200 = pos_table[:SEQ_LEN]
    posx = jnp.concatenate([pos200, pos200], axis=1)
    out_t = _run(xt, wt2, posx)
    return jnp.transpose(out_t, (2, 0, 1))
